# baseline (speedup 1.0000x reference)
import functools

import jax
import jax.numpy as jnp
from jax import lax
from jax.experimental import pallas as pl
from jax.experimental.pallas import tpu as pltpu
from jax.experimental.pallas import tpu_sc as plsc

N = 10000
E = 320000
D = 128

NC = 2
NS = 16
NW = NC * NS
C = 128
K = 80
EPT = K * C
EPAD = NW * EPT
DUMMY = N
SP_ROWS = 10112
RPT = SP_ROWS // NS

_mesh = plsc.VectorSubcoreMesh(core_axis_name="c", subcore_axis_name="s")


def _deg_body(dst_hbm, zrows_hbm, ones_hbm, out_hbm, didx_v, ones_v, degbuf_sp):
    c = lax.axis_index("c")
    s = lax.axis_index("s")
    tid = c * NS + s

    pltpu.sync_copy(ones_hbm, ones_v)

    pltpu.sync_copy(zrows_hbm, degbuf_sp.at[pl.ds(s * RPT, RPT)])
    plsc.subcore_barrier()

    pltpu.sync_copy(dst_hbm.at[pl.ds(tid * K, K)], didx_v)

    def step(k, _):
        pltpu.sync_copy(ones_v, degbuf_sp.at[didx_v.at[k]], add=True)
        return _

    lax.fori_loop(0, K, step, None)
    plsc.subcore_barrier()

    pltpu.sync_copy(degbuf_sp.at[pl.ds(s * RPT, RPT)],
                    out_hbm.at[c, pl.ds(s * RPT, RPT)])


_deg_call = pl.kernel(
    _deg_body,
    out_type=jax.ShapeDtypeStruct((NC, SP_ROWS, D), jnp.float32),
    mesh=_mesh,
    scratch_types=[
        pltpu.VMEM((K, C), jnp.int32),
        pltpu.VMEM((C, D), jnp.float32),
        pltpu.VMEM_SHARED((SP_ROWS, D), jnp.float32),
    ],
)


_H = 8
_NSLABS = EPAD // (C * _H)
_Q0, _Q1 = 20, 0
_DEPTH = 2
assert (_Q0 + _Q1) * NS == _NSLABS
assert _H % _DEPTH == 0


def _agg_body(g_hbm, src_hbm, dst_hbm, zrows_hbm, out_hbm,
              sidx_v, didx_v, *scr):
    rows = scr[:_DEPTH]
    aggbuf_sp = scr[_DEPTH]
    sems = scr[_DEPTH + 1:]
    c = lax.axis_index("c")
    s = lax.axis_index("s")

    pltpu.sync_copy(zrows_hbm, aggbuf_sp.at[pl.ds(s * RPT, RPT)])
    plsc.subcore_barrier()

    nslab = jnp.where(c == 0, _Q0, _Q1)
    slab0 = jnp.where(c == 0, s * _Q0, NS * _Q0 + s * _Q1)

    for h in range(max(_Q0, _Q1)):
        @pl.when(h < nslab)
        def _slab():
            base_row = (slab0 + h) * _H
            pltpu.sync_copy(src_hbm.at[pl.ds(base_row, _H)], sidx_v)
            pltpu.sync_copy(dst_hbm.at[pl.ds(base_row, _H)], didx_v)
            for j in range(_DEPTH):
                pltpu.async_copy(g_hbm.at[sidx_v.at[j]], rows[j], sems[j])

            def group(p, _):
                base = p * _DEPTH
                for j in range(_DEPTH):
                    k = base + j
                    pltpu.make_async_copy(
                        g_hbm.at[sidx_v.at[k]], rows[j], sems[j]).wait()
                    pltpu.sync_copy(
                        rows[j], aggbuf_sp.at[didx_v.at[k]], add=True)

                    @pl.when(k + _DEPTH < _H)
                    def _issue():
                        pltpu.async_copy(
                            g_hbm.at[sidx_v.at[k + _DEPTH]], rows[j], sems[j])

                return _

            lax.fori_loop(0, _H // _DEPTH, group, None)

    plsc.subcore_barrier()
    pltpu.sync_copy(aggbuf_sp.at[pl.ds(s * RPT, RPT)],
                    out_hbm.at[c, pl.ds(s * RPT, RPT)])


_agg_call = pl.kernel(
    _agg_body,
    out_type=jax.ShapeDtypeStruct((NC, SP_ROWS, D), jnp.float32),
    mesh=_mesh,
    scratch_types=[
        pltpu.VMEM((_H, C), jnp.int32),
        pltpu.VMEM((_H, C), jnp.int32),
    ] + [pltpu.VMEM((C, D), jnp.float32)] * _DEPTH + [
        pltpu.VMEM_SHARED((SP_ROWS, D), jnp.float32),
    ] + [pltpu.SemaphoreType.DMA] * _DEPTH,
)


_RB = 2000
_GRID = N // _RB


def _dinv_of(degp_blk):
    deg = 1.0 + degp_blk[0, :, 0:1] + degp_blk[1, :, 0:1]
    return lax.rsqrt(deg)


def _pre_body(x_ref, w_ref, degp_ref, g_ref):
    dinv = _dinv_of(degp_ref[...])
    g_ref[...] = jnp.dot(x_ref[...], w_ref[...],
                         preferred_element_type=jnp.float32) * dinv


def _mid_body(aggp_ref, g_ref, degp_ref, b_ref, w_ref, gn_ref):
    dinv = _dinv_of(degp_ref[...])
    a = aggp_ref[0] + aggp_ref[1] + g_ref[...]
    h = jnp.maximum(dinv * a + b_ref[...], 0.0)
    gn_ref[...] = jnp.dot(h, w_ref[...],
                          preferred_element_type=jnp.float32) * dinv


def _post_body(aggp_ref, g_ref, degp_ref, b_ref, x_ref, o_ref):
    dinv = _dinv_of(degp_ref[...])
    a = aggp_ref[0] + aggp_ref[1] + g_ref[...]
    o_ref[...] = dinv * a + b_ref[...] + x_ref[...]


_spec_rows = pl.BlockSpec((_RB, D), lambda i: (i, 0))
_spec_w = pl.BlockSpec((D, D), lambda i: (0, 0))
_spec_b = pl.BlockSpec((1, D), lambda i: (0, 0))
_spec_degp = pl.BlockSpec((NC, _RB, D), lambda i: (0, i, 0))
_spec_aggp = pl.BlockSpec((NC, _RB, D), lambda i: (0, i, 0))
_out_rows = jax.ShapeDtypeStruct((N, D), jnp.float32)

_pre_call = pl.pallas_call(
    _pre_body, grid=(_GRID,),
    in_specs=[_spec_rows, _spec_w, _spec_degp],
    out_specs=_spec_rows, out_shape=_out_rows)

_mid_call = pl.pallas_call(
    _mid_body, grid=(_GRID,),
    in_specs=[_spec_aggp, _spec_rows, _spec_degp, _spec_b, _spec_w],
    out_specs=_spec_rows, out_shape=_out_rows)

_post_call = pl.pallas_call(
    _post_body, grid=(_GRID,),
    in_specs=[_spec_aggp, _spec_rows, _spec_degp, _spec_b, _spec_rows],
    out_specs=_spec_rows, out_shape=_out_rows)


def kernel(x, edge_index, W1, b1, W2, b2, W3, b3):
    pad = EPAD - E
    src = jnp.concatenate(
        [edge_index[0], jnp.zeros((pad,), jnp.int32)]).reshape(NW * K, C)
    dst = jnp.concatenate(
        [edge_index[1], jnp.full((pad,), DUMMY, jnp.int32)]).reshape(NW * K, C)
    zD = jnp.zeros((RPT, D), jnp.float32)
    b1r, b2r, b3r = (b.reshape(1, D) for b in (b1, b2, b3))

    degp = _deg_call(dst, zD, jnp.ones((C, D), jnp.float32))
    g1 = _pre_call(x, W1, degp)
    aggp1 = _agg_call(g1, src, dst, zD)
    g2 = _mid_call(aggp1, g1, degp, b1r, W2)
    aggp2 = _agg_call(g2, src, dst, zD)
    g3 = _mid_call(aggp2, g2, degp, b2r, W3)
    aggp3 = _agg_call(g3, src, dst, zD)
    return _post_call(aggp3, g3, degp, b3r, x)


# speedup vs baseline: 1.4783x; 1.4783x over previous
import functools

import jax
import jax.numpy as jnp
from jax import lax
from jax.experimental import pallas as pl
from jax.experimental.pallas import tpu as pltpu
from jax.experimental.pallas import tpu_sc as plsc

N = 10000
E = 320000
D = 128

NC = 2
NS = 16
NW = NC * NS
C = 128
K = 80
EPT = K * C
EPAD = NW * EPT
DUMMY = N
SP_ROWS = 10112
RPT = SP_ROWS // NS

_mesh = plsc.VectorSubcoreMesh(core_axis_name="c", subcore_axis_name="s")


def _deg_body(dst_hbm, zrows_hbm, ones_hbm, out_hbm, didx_v, ones_v, degbuf_sp):
    c = lax.axis_index("c")
    s = lax.axis_index("s")
    tid = c * NS + s

    pltpu.sync_copy(ones_hbm, ones_v)

    pltpu.sync_copy(zrows_hbm, degbuf_sp.at[pl.ds(s * RPT, RPT)])
    plsc.subcore_barrier()

    pltpu.sync_copy(dst_hbm.at[pl.ds(tid * K, K)], didx_v)

    def step(k, _):
        pltpu.sync_copy(ones_v, degbuf_sp.at[didx_v.at[k]], add=True)
        return _

    lax.fori_loop(0, K, step, None)
    plsc.subcore_barrier()

    pltpu.sync_copy(degbuf_sp.at[pl.ds(s * RPT, RPT)],
                    out_hbm.at[c, pl.ds(s * RPT, RPT)])


_deg_call = pl.kernel(
    _deg_body,
    out_type=jax.ShapeDtypeStruct((NC, SP_ROWS, D), jnp.float32),
    mesh=_mesh,
    scratch_types=[
        pltpu.VMEM((K, C), jnp.int32),
        pltpu.VMEM((C, D), jnp.float32),
        pltpu.VMEM_SHARED((SP_ROWS, D), jnp.float32),
    ],
)


_H = 8
_NSLABS = EPAD // (C * _H)
_Q0, _Q1 = 19, 1
_DEPTH = 2
assert (_Q0 + _Q1) * NS == _NSLABS
assert _H % _DEPTH == 0


def _agg_body(g_hbm, src_hbm, dst_hbm, zrows_hbm, out_hbm,
              sidx_v, didx_v, *scr):
    rows = scr[:_DEPTH]
    aggbuf_sp = scr[_DEPTH]
    sems = scr[_DEPTH + 1:]
    c = lax.axis_index("c")
    s = lax.axis_index("s")

    pltpu.sync_copy(zrows_hbm, aggbuf_sp.at[pl.ds(s * RPT, RPT)])
    plsc.subcore_barrier()

    nslab = jnp.where(c == 0, _Q0, _Q1)
    slab0 = jnp.where(c == 0, s * _Q0, NS * _Q0 + s * _Q1)

    for h in range(max(_Q0, _Q1)):
        @pl.when(h < nslab)
        def _slab():
            base_row = (slab0 + h) * _H
            pltpu.sync_copy(src_hbm.at[pl.ds(base_row, _H)], sidx_v)
            pltpu.sync_copy(dst_hbm.at[pl.ds(base_row, _H)], didx_v)
            for j in range(_DEPTH):
                pltpu.async_copy(g_hbm.at[sidx_v.at[j]], rows[j], sems[j])

            def group(p, _):
                base = p * _DEPTH
                for j in range(_DEPTH):
                    k = base + j
                    pltpu.make_async_copy(
                        g_hbm.at[sidx_v.at[k]], rows[j], sems[j]).wait()
                    pltpu.sync_copy(
                        rows[j], aggbuf_sp.at[didx_v.at[k]], add=True)

                    @pl.when(k + _DEPTH < _H)
                    def _issue():
                        pltpu.async_copy(
                            g_hbm.at[sidx_v.at[k + _DEPTH]], rows[j], sems[j])

                return _

            lax.fori_loop(0, _H // _DEPTH, group, None)

    plsc.subcore_barrier()
    pltpu.sync_copy(aggbuf_sp.at[pl.ds(s * RPT, RPT)],
                    out_hbm.at[c, pl.ds(s * RPT, RPT)])


_agg_call = pl.kernel(
    _agg_body,
    out_type=jax.ShapeDtypeStruct((NC, SP_ROWS, D), jnp.float32),
    mesh=_mesh,
    scratch_types=[
        pltpu.VMEM((_H, C), jnp.int32),
        pltpu.VMEM((_H, C), jnp.int32),
    ] + [pltpu.VMEM((C, D), jnp.float32)] * _DEPTH + [
        pltpu.VMEM_SHARED((SP_ROWS, D), jnp.float32),
    ] + [pltpu.SemaphoreType.DMA] * _DEPTH,
)


_RB = 2000
_GRID = N // _RB


def _dinv_of(degp_blk):
    deg = 1.0 + degp_blk[0, :, 0:1] + degp_blk[1, :, 0:1]
    return lax.rsqrt(deg)


def _pre_body(x_ref, w_ref, degp_ref, g_ref):
    dinv = _dinv_of(degp_ref[...])
    g_ref[...] = jnp.dot(x_ref[...], w_ref[...],
                         preferred_element_type=jnp.float32) * dinv


def _mid_body(aggp_ref, g_ref, degp_ref, b_ref, w_ref, gn_ref):
    dinv = _dinv_of(degp_ref[...])
    a = aggp_ref[0] + aggp_ref[1] + g_ref[...]
    h = jnp.maximum(dinv * a + b_ref[...], 0.0)
    gn_ref[...] = jnp.dot(h, w_ref[...],
                          preferred_element_type=jnp.float32) * dinv


def _post_body(aggp_ref, g_ref, degp_ref, b_ref, x_ref, o_ref):
    dinv = _dinv_of(degp_ref[...])
    a = aggp_ref[0] + aggp_ref[1] + g_ref[...]
    o_ref[...] = dinv * a + b_ref[...] + x_ref[...]


_spec_rows = pl.BlockSpec((_RB, D), lambda i: (i, 0))
_spec_w = pl.BlockSpec((D, D), lambda i: (0, 0))
_spec_b = pl.BlockSpec((1, D), lambda i: (0, 0))
_spec_degp = pl.BlockSpec((NC, _RB, D), lambda i: (0, i, 0))
_spec_aggp = pl.BlockSpec((NC, _RB, D), lambda i: (0, i, 0))
_out_rows = jax.ShapeDtypeStruct((N, D), jnp.float32)

_pre_call = pl.pallas_call(
    _pre_body, grid=(_GRID,),
    in_specs=[_spec_rows, _spec_w, _spec_degp],
    out_specs=_spec_rows, out_shape=_out_rows)

_mid_call = pl.pallas_call(
    _mid_body, grid=(_GRID,),
    in_specs=[_spec_aggp, _spec_rows, _spec_degp, _spec_b, _spec_w],
    out_specs=_spec_rows, out_shape=_out_rows)

_post_call = pl.pallas_call(
    _post_body, grid=(_GRID,),
    in_specs=[_spec_aggp, _spec_rows, _spec_degp, _spec_b, _spec_rows],
    out_specs=_spec_rows, out_shape=_out_rows)


def kernel(x, edge_index, W1, b1, W2, b2, W3, b3):
    pad = EPAD - E
    src = jnp.concatenate(
        [edge_index[0], jnp.zeros((pad,), jnp.int32)]).reshape(NW * K, C)
    dst = jnp.concatenate(
        [edge_index[1], jnp.full((pad,), DUMMY, jnp.int32)]).reshape(NW * K, C)
    zD = jnp.zeros((RPT, D), jnp.float32)
    b1r, b2r, b3r = (b.reshape(1, D) for b in (b1, b2, b3))

    degp = _deg_call(dst, zD, jnp.ones((C, D), jnp.float32))
    g1 = _pre_call(x, W1, degp)
    aggp1 = _agg_call(g1, src, dst, zD)
    g2 = _mid_call(aggp1, g1, degp, b1r, W2)
    aggp2 = _agg_call(g2, src, dst, zD)
    g3 = _mid_call(aggp2, g2, degp, b2r, W3)
    aggp3 = _agg_call(g3, src, dst, zD)
    return _post_call(aggp3, g3, degp, b3r, x)


# speedup vs baseline: 1.4788x; 1.0004x over previous
import jax
import jax.numpy as jnp
from jax import lax
from jax.experimental import pallas as pl
from jax.experimental.pallas import tpu as pltpu
from jax.experimental.pallas import tpu_sc as plsc

N = 10000
E = 320000
D = 128

NC = 2
NS = 16
NW = NC * NS
C = 128
K = 80
EPT = K * C
EPAD = NW * EPT
DUMMY = N
SP_ROWS = 10112
RPT = SP_ROWS // NS

_mesh = plsc.VectorSubcoreMesh(core_axis_name="c", subcore_axis_name="s")


def _deg_body(dst_hbm, zrows_hbm, ones_hbm, out_hbm, didx_v, ones_v, degbuf_sp):
    c = lax.axis_index("c")
    s = lax.axis_index("s")
    tid = c * NS + s

    pltpu.sync_copy(ones_hbm, ones_v)

    pltpu.sync_copy(zrows_hbm, degbuf_sp.at[pl.ds(s * RPT, RPT)])
    plsc.subcore_barrier()

    pltpu.sync_copy(dst_hbm.at[pl.ds(tid * K, K)], didx_v)

    def step(k, _):
        pltpu.sync_copy(ones_v, degbuf_sp.at[didx_v.at[k]], add=True)
        return _

    lax.fori_loop(0, K, step, None)
    plsc.subcore_barrier()

    pltpu.sync_copy(degbuf_sp.at[pl.ds(s * RPT, RPT)],
                    out_hbm.at[c, pl.ds(s * RPT, RPT)])


_deg_call = pl.kernel(
    _deg_body,
    out_type=jax.ShapeDtypeStruct((NC, SP_ROWS, D), jnp.float32),
    mesh=_mesh,
    scratch_types=[
        pltpu.VMEM((K, C), jnp.int32),
        pltpu.VMEM((C, D), jnp.float32),
        pltpu.VMEM_SHARED((SP_ROWS, D), jnp.float32),
    ],
)


_H = 8
_NSLABS = EPAD // (C * _H)
_Q0, _Q1 = 19, 1
_DEPTH = 2
assert (_Q0 + _Q1) * NS == _NSLABS
assert _H % _DEPTH == 0


def _agg_body(g_hbm, src_hbm, dst_hbm, zrows_hbm, out_hbm,
              sidx_v, didx_v, *scr):
    rows = scr[:_DEPTH]
    aggbuf_sp = scr[_DEPTH]
    sems = scr[_DEPTH + 1:]
    c = lax.axis_index("c")
    s = lax.axis_index("s")

    pltpu.sync_copy(zrows_hbm, aggbuf_sp.at[pl.ds(s * RPT, RPT)])
    plsc.subcore_barrier()

    nslab = jnp.where(c == 0, _Q0, _Q1)
    slab0 = jnp.where(c == 0, s * _Q0, NS * _Q0 + s * _Q1)

    for h in range(max(_Q0, _Q1)):
        @pl.when(h < nslab)
        def _slab():
            base_row = (slab0 + h) * _H
            pltpu.sync_copy(src_hbm.at[pl.ds(base_row, _H)], sidx_v)
            pltpu.sync_copy(dst_hbm.at[pl.ds(base_row, _H)], didx_v)
            for j in range(_DEPTH):
                pltpu.async_copy(g_hbm.at[sidx_v.at[j]], rows[j], sems[j])

            def group(p, _):
                base = p * _DEPTH
                for j in range(_DEPTH):
                    k = base + j
                    pltpu.make_async_copy(
                        g_hbm.at[sidx_v.at[k]], rows[j], sems[j]).wait()
                    pltpu.sync_copy(
                        rows[j], aggbuf_sp.at[didx_v.at[k]], add=True)

                    @pl.when(k + _DEPTH < _H)
                    def _issue():
                        pltpu.async_copy(
                            g_hbm.at[sidx_v.at[k + _DEPTH]], rows[j], sems[j])

                return _

            lax.fori_loop(0, _H // _DEPTH, group, None)

    plsc.subcore_barrier()
    pltpu.sync_copy(aggbuf_sp.at[pl.ds(s * RPT, RPT)],
                    out_hbm.at[c, pl.ds(s * RPT, RPT)])


_agg_call = pl.kernel(
    _agg_body,
    out_type=jax.ShapeDtypeStruct((NC, SP_ROWS, D), jnp.float32),
    mesh=_mesh,
    scratch_types=[
        pltpu.VMEM((_H, C), jnp.int32),
        pltpu.VMEM((_H, C), jnp.int32),
    ] + [pltpu.VMEM((C, D), jnp.float32)] * _DEPTH + [
        pltpu.VMEM_SHARED((SP_ROWS, D), jnp.float32),
    ] + [pltpu.SemaphoreType.DMA] * _DEPTH,
)


_RB = 2000
_GRID = N // _RB


def _dinv_of(degp_blk):
    deg = 1.0 + degp_blk[0, :, 0:1] + degp_blk[1, :, 0:1]
    return lax.rsqrt(deg)


def _pre_body(x_ref, w_ref, degp_ref, g_ref):
    dinv = _dinv_of(degp_ref[...])
    g_ref[...] = jnp.dot(x_ref[...], w_ref[...],
                         preferred_element_type=jnp.float32) * dinv


def _mid_body(aggp_ref, g_ref, degp_ref, b_ref, w_ref, gn_ref):
    dinv = _dinv_of(degp_ref[...])
    a = aggp_ref[0] + aggp_ref[1] + g_ref[...]
    h = jnp.maximum(dinv * a + b_ref[...], 0.0)
    gn_ref[...] = jnp.dot(h, w_ref[...],
                          preferred_element_type=jnp.float32) * dinv


def _post_body(aggp_ref, g_ref, degp_ref, b_ref, x_ref, o_ref):
    dinv = _dinv_of(degp_ref[...])
    a = aggp_ref[0] + aggp_ref[1] + g_ref[...]
    o_ref[...] = dinv * a + b_ref[...] + x_ref[...]


_spec_rows = pl.BlockSpec((_RB, D), lambda i: (i, 0))
_spec_w = pl.BlockSpec((D, D), lambda i: (0, 0))
_spec_b = pl.BlockSpec((1, D), lambda i: (0, 0))
_spec_degp = pl.BlockSpec((NC, _RB, D), lambda i: (0, i, 0))
_spec_aggp = pl.BlockSpec((NC, _RB, D), lambda i: (0, i, 0))
_out_rows = jax.ShapeDtypeStruct((N, D), jnp.float32)

_pre_call = pl.pallas_call(
    _pre_body, grid=(_GRID,),
    in_specs=[_spec_rows, _spec_w, _spec_degp],
    out_specs=_spec_rows, out_shape=_out_rows)

_mid_call = pl.pallas_call(
    _mid_body, grid=(_GRID,),
    in_specs=[_spec_aggp, _spec_rows, _spec_degp, _spec_b, _spec_w],
    out_specs=_spec_rows, out_shape=_out_rows)

_post_call = pl.pallas_call(
    _post_body, grid=(_GRID,),
    in_specs=[_spec_aggp, _spec_rows, _spec_degp, _spec_b, _spec_rows],
    out_specs=_spec_rows, out_shape=_out_rows)


def kernel(x, edge_index, W1, b1, W2, b2, W3, b3):
    pad = EPAD - E
    src = jnp.concatenate(
        [edge_index[0], jnp.zeros((pad,), jnp.int32)]).reshape(NW * K, C)
    dst = jnp.concatenate(
        [edge_index[1], jnp.full((pad,), DUMMY, jnp.int32)]).reshape(NW * K, C)
    zD = jnp.zeros((RPT, D), jnp.float32)
    b1r, b2r, b3r = (b.reshape(1, D) for b in (b1, b2, b3))

    degp = _deg_call(dst, zD, jnp.ones((C, D), jnp.float32))
    g1 = _pre_call(x, W1, degp)
    aggp1 = _agg_call(g1, src, dst, zD)
    g2 = _mid_call(aggp1, g1, degp, b1r, W2)
    aggp2 = _agg_call(g2, src, dst, zD)
    g3 = _mid_call(aggp2, g2, degp, b2r, W3)
    aggp3 = _agg_call(g3, src, dst, zD)
    return _post_call(aggp3, g3, degp, b3r, x)
